# baseline (speedup 1.0000x reference)
import functools

import jax
import jax.numpy as jnp
from jax import lax
from jax.experimental import pallas as pl
from jax.experimental.pallas import tpu as pltpu
from jax.experimental.pallas import tpu_sc as plsc

N_NODES = 10000
D_FEAT = 256
N_GRAPHS = 64

NC = 2
NS = 16
NW = NC * NS

SPAN = N_NODES // NW
TAIL = N_NODES - SPAN * NW
HALF1 = 160
NGRP1 = HALF1 // 16
IDS_PAD = 10240


def _sc_segment_sum(x, batch_i32):
  mesh = plsc.VectorSubcoreMesh(core_axis_name="c", subcore_axis_name="s")

  @functools.partial(
      pl.kernel,
      out_type=jax.ShapeDtypeStruct((NW, N_GRAPHS, D_FEAT), jnp.float32),
      mesh=mesh,
      scratch_types=[
          pltpu.VMEM((SPAN + TAIL + 8, D_FEAT), jnp.float32),
          pltpu.VMEM((SPAN + TAIL + 16,), jnp.int32),
          pltpu.VMEM((N_GRAPHS, D_FEAT), jnp.float32),
          pltpu.SemaphoreType.DMA,
          pltpu.SemaphoreType.DMA,
          pltpu.SemaphoreType.DMA,
      ],
  )
  def k(x_hbm, ids_hbm, sums_hbm, rows_v, idx_v, acc_v, sem1, sem2, semt):
    c = lax.axis_index("c")
    s = lax.axis_index("s")
    wid = s * NC + c
    base = wid * SPAN
    HALF2 = SPAN - HALF1

    pltpu.async_copy(
        x_hbm.at[pl.ds(base, HALF1)], rows_v.at[pl.ds(0, HALF1)], sem1)
    pltpu.async_copy(
        ids_hbm.at[pl.ds(base, HALF1)], idx_v.at[pl.ds(0, HALF1)], sem1)
    pltpu.async_copy(
        x_hbm.at[pl.ds(base + HALF1, HALF2)],
        rows_v.at[pl.ds(HALF1, HALF2)], sem2)
    pltpu.async_copy(
        ids_hbm.at[pl.ds(base + HALF1, HALF2)],
        idx_v.at[pl.ds(HALF1, HALF2)], sem2)

    @pl.when(wid == NW - 1)
    def _():
      pltpu.async_copy(
          x_hbm.at[pl.ds(NW * SPAN, TAIL)], rows_v.at[pl.ds(SPAN, TAIL)], semt)
      pltpu.async_copy(
          ids_hbm.at[pl.ds(NW * SPAN, TAIL)], idx_v.at[pl.ds(SPAN, TAIL)], semt)

    zero = jnp.zeros((16,), jnp.float32)

    def zrow(r, carry):
      for j in range(D_FEAT // 16):
        acc_v[r, pl.ds(16 * j, 16)] = zero
      return carry

    lax.fori_loop(0, N_GRAPHS, zrow, 0)

    pltpu.make_async_copy(
        x_hbm.at[pl.ds(0, HALF1)], rows_v.at[pl.ds(0, HALF1)], sem1).wait()
    pltpu.make_async_copy(
        ids_hbm.at[pl.ds(0, HALF1)], idx_v.at[pl.ds(0, HALF1)], sem1).wait()

    def rowgroup(t, carry):
      gvec = idx_v[pl.ds(16 * t, 16)]
      g0 = gvec[0]

      @pl.when(g0 == gvec[15])
      def _():
        for j in range(D_FEAT // 16):
          sl = pl.ds(16 * j, 16)
          v = [rows_v[16 * t + l, sl] for l in range(16)]
          while len(v) > 1:
            v = [a + b for a, b in zip(v[::2], v[1::2])]
          acc_v[g0, sl] = acc_v[g0, sl] + v[0]

      @pl.when(g0 != gvec[15])
      def _():
        for l in range(16):
          g = gvec[l]
          r = 16 * t + l
          for j in range(D_FEAT // 16):
            sl = pl.ds(16 * j, 16)
            acc_v[g, sl] = acc_v[g, sl] + rows_v[r, sl]

      return carry

    lax.fori_loop(0, NGRP1, rowgroup, 0)

    pltpu.make_async_copy(
        x_hbm.at[pl.ds(0, HALF2)], rows_v.at[pl.ds(HALF1, HALF2)], sem2).wait()
    pltpu.make_async_copy(
        ids_hbm.at[pl.ds(0, HALF2)], idx_v.at[pl.ds(HALF1, HALF2)], sem2).wait()

    @pl.when(wid == NW - 1)
    def _():
      pltpu.make_async_copy(
          x_hbm.at[pl.ds(0, TAIL)], rows_v.at[pl.ds(SPAN, TAIL)], semt).wait()
      pltpu.make_async_copy(
          ids_hbm.at[pl.ds(0, TAIL)], idx_v.at[pl.ds(SPAN, TAIL)], semt).wait()

    r_pad = jnp.where(wid == NW - 1, SPAN + TAIL, SPAN)
    ngroups = jnp.where(wid == NW - 1, (SPAN + TAIL + 8) // 16, (SPAN + 8) // 16)
    glast = idx_v[pl.ds(r_pad - 16, 16)]
    idx_v[pl.ds(r_pad, 16)] = jnp.full((16,), glast[15], jnp.int32)
    for l in range(8):
      for j in range(D_FEAT // 16):
        rows_v[r_pad + l, pl.ds(16 * j, 16)] = zero

    lax.fori_loop(NGRP1, ngroups, rowgroup, 0)

    pltpu.sync_copy(acc_v, sums_hbm.at[wid])

  return k(x, batch_i32)


def _tc_mlp(sums2, ids_pad, u, W1, b1, W2, b2):
  def body(sums_ref, ids_ref, u_ref, W1_ref, b1_ref, W2_ref, b2_ref, o_ref):
    sums = jnp.sum(sums_ref[...], axis=0)
    gid = lax.broadcasted_iota(jnp.int32, (N_GRAPHS, 1), 0)
    eq = (ids_ref[...] == gid).astype(jnp.float32)
    cnt = jnp.sum(eq, axis=1, keepdims=True)
    mean = sums / jnp.maximum(cnt, 1.0)
    d_g = u_ref.shape[1]
    z = (
        jnp.dot(u_ref[...], W1_ref[0:d_g, :], preferred_element_type=jnp.float32)
        + jnp.dot(mean, W1_ref[d_g:, :], preferred_element_type=jnp.float32)
        + b1_ref[...]
    )
    h = jnp.where(z > 0, z, jnp.exp(jnp.minimum(z, 0.0)) - 1.0)
    o_ref[...] = (
        jnp.dot(h, W2_ref[...], preferred_element_type=jnp.float32) + b2_ref[...]
    )

  return pl.pallas_call(
      body,
      out_shape=jax.ShapeDtypeStruct((u.shape[0], W2.shape[1]), jnp.float32),
  )(sums2, ids_pad, u, W1, b1.reshape(1, -1), W2, b2.reshape(1, -1))


def kernel(x, edge_index, edge_attr, u, batch, W1, b1, W2, b2):
  del edge_index, edge_attr
  batch_i32 = batch.astype(jnp.int32)
  sums32 = _sc_segment_sum(x, batch_i32)
  return _tc_mlp(sums32, batch_i32.reshape(1, -1), u, W1, b1, W2, b2)


# speedup vs baseline: 1.0959x; 1.0959x over previous
import functools

import jax
import jax.numpy as jnp
from jax import lax
from jax.experimental import pallas as pl
from jax.experimental.pallas import tpu as pltpu
from jax.experimental.pallas import tpu_sc as plsc

N_NODES = 10000
D_FEAT = 256
N_GRAPHS = 64

NC = 2
NS = 16
NW = NC * NS

SPAN = N_NODES // NW
NGRP = SPAN // 16
REM = SPAN - NGRP * 16
TAIL = N_NODES - SPAN * NW
IDS_PAD = 10240


def _sc_segment_sum(x, batch_i32):
  mesh = plsc.VectorSubcoreMesh(core_axis_name="c", subcore_axis_name="s")

  @functools.partial(
      pl.kernel,
      out_type=jax.ShapeDtypeStruct((NW, N_GRAPHS, D_FEAT), jnp.float32),
      mesh=mesh,
      scratch_types=[
          pltpu.VMEM((SPAN + TAIL + 8, D_FEAT), jnp.float32),
          pltpu.VMEM((SPAN + TAIL + 16,), jnp.int32),
          pltpu.VMEM((N_GRAPHS, D_FEAT), jnp.float32),
          pltpu.SemaphoreType.DMA,
          pltpu.SemaphoreType.DMA,
          pltpu.SemaphoreType.DMA,
      ],
  )
  def k(x_hbm, ids_hbm, sums_hbm, rows_v, idx_v, acc_v, sem1, sem2, semt):
    c = lax.axis_index("c")
    s = lax.axis_index("s")
    wid = s * NC + c
    base = wid * SPAN

    pltpu.async_copy(
        x_hbm.at[pl.ds(base, SPAN)], rows_v.at[pl.ds(0, SPAN)], sem1)
    pltpu.async_copy(
        ids_hbm.at[pl.ds(base, SPAN)], idx_v.at[pl.ds(0, SPAN)], sem1)

    @pl.when(wid == NW - 1)
    def _():
      pltpu.async_copy(
          x_hbm.at[pl.ds(NW * SPAN, TAIL)], rows_v.at[pl.ds(SPAN, TAIL)], semt)
      pltpu.async_copy(
          ids_hbm.at[pl.ds(NW * SPAN, TAIL)], idx_v.at[pl.ds(SPAN, TAIL)], semt)

    zero = jnp.zeros((16,), jnp.float32)

    def zrow(r, carry):
      for j in range(D_FEAT // 16):
        acc_v[r, pl.ds(16 * j, 16)] = zero
      return carry

    lax.fori_loop(0, N_GRAPHS, zrow, 0)

    pltpu.make_async_copy(
        x_hbm.at[pl.ds(0, SPAN)], rows_v.at[pl.ds(0, SPAN)], sem1).wait()
    pltpu.make_async_copy(
        ids_hbm.at[pl.ds(0, SPAN)], idx_v.at[pl.ds(0, SPAN)], sem1).wait()

    @pl.when(wid == NW - 1)
    def _():
      pltpu.make_async_copy(
          x_hbm.at[pl.ds(0, TAIL)], rows_v.at[pl.ds(SPAN, TAIL)], semt).wait()
      pltpu.make_async_copy(
          ids_hbm.at[pl.ds(0, TAIL)], idx_v.at[pl.ds(SPAN, TAIL)], semt).wait()

    def rowgroup(t, carry):
      gvec = idx_v[pl.ds(16 * t, 16)]
      g0 = gvec[0]

      @pl.when(g0 == gvec[15])
      def _():
        for j in range(D_FEAT // 16):
          sl = pl.ds(16 * j, 16)
          v = [rows_v[16 * t + l, sl] for l in range(16)]
          while len(v) > 1:
            v = [a + b for a, b in zip(v[::2], v[1::2])]
          acc_v[g0, sl] = acc_v[g0, sl] + v[0]

      @pl.when(g0 != gvec[15])
      def _():
        for l in range(16):
          g = gvec[l]
          r = 16 * t + l
          for j in range(D_FEAT // 16):
            sl = pl.ds(16 * j, 16)
            acc_v[g, sl] = acc_v[g, sl] + rows_v[r, sl]

      return carry

    lax.fori_loop(0, NGRP, rowgroup, 0)

    nrem = jnp.where(wid == NW - 1, REM + TAIL, REM)

    def rowrem(r, carry):
      gv = idx_v[pl.ds(r, 16)]
      g = gv[0]
      for j in range(D_FEAT // 16):
        sl = pl.ds(16 * j, 16)
        acc_v[g, sl] = acc_v[g, sl] + rows_v[r, sl]
      return carry

    lax.fori_loop(NGRP * 16, NGRP * 16 + nrem, rowrem, 0)

    pltpu.sync_copy(acc_v, sums_hbm.at[wid])

  return k(x, batch_i32)


def _tc_mlp(sums2, ids_pad, u, W1, b1, W2, b2):
  def body(sums_ref, ids_ref, u_ref, W1_ref, b1_ref, W2_ref, b2_ref, o_ref):
    sums = jnp.sum(sums_ref[...], axis=0)
    gid = lax.broadcasted_iota(jnp.int32, (N_GRAPHS, 1), 0)
    eq = (ids_ref[...] == gid).astype(jnp.float32)
    cnt = jnp.sum(eq, axis=1, keepdims=True)
    mean = sums / jnp.maximum(cnt, 1.0)
    d_g = u_ref.shape[1]
    z = (
        jnp.dot(u_ref[...], W1_ref[0:d_g, :], preferred_element_type=jnp.float32)
        + jnp.dot(mean, W1_ref[d_g:, :], preferred_element_type=jnp.float32)
        + b1_ref[...]
    )
    h = jnp.where(z > 0, z, jnp.exp(jnp.minimum(z, 0.0)) - 1.0)
    o_ref[...] = (
        jnp.dot(h, W2_ref[...], preferred_element_type=jnp.float32) + b2_ref[...]
    )

  return pl.pallas_call(
      body,
      out_shape=jax.ShapeDtypeStruct((u.shape[0], W2.shape[1]), jnp.float32),
  )(sums2, ids_pad, u, W1, b1.reshape(1, -1), W2, b2.reshape(1, -1))


def kernel(x, edge_index, edge_attr, u, batch, W1, b1, W2, b2):
  del edge_index, edge_attr
  batch_i32 = batch.astype(jnp.int32)
  sums32 = _sc_segment_sum(x, batch_i32)
  return _tc_mlp(sums32, batch_i32.reshape(1, -1), u, W1, b1, W2, b2)
